# baseline (speedup 1.0000x reference)
import jax
import jax.numpy as jnp
from jax.experimental import pallas as pl
from jax.experimental.pallas import tpu as pltpu

B, N, D = 32, 576, 384


def _body(x_ref, p_ref, o_ref):
    o_ref[...] = x_ref[...] + p_ref[...]


def kernel(inputs, pos_table):
    return pl.pallas_call(
        _body,
        grid=(3, B),
        in_specs=[
            pl.BlockSpec((1, N, 128), lambda d, b: (b, 0, d)),
            pl.BlockSpec((N, 128), lambda d, b: (0, d)),
        ],
        out_specs=pl.BlockSpec((1, N, 128), lambda d, b: (b, 0, d)),
        out_shape=jax.ShapeDtypeStruct((B, N, D), jnp.float32),
    )(inputs, pos_table)


# speedup vs baseline: 1.3716x; 1.3716x over previous
import jax
import jax.numpy as jnp
from jax.experimental import pallas as pl
from jax.experimental.pallas import tpu as pltpu

B, N, D = 32, 576, 384


def _body(x_ref, p_ref, o_ref):
    o_ref[...] = x_ref[...] + p_ref[...]


def kernel(inputs, pos_table):
    return pl.pallas_call(
        _body,
        grid=(2, B),
        in_specs=[
            pl.BlockSpec((1, N // 2, D), lambda i, b: (b, i, 0)),
            pl.BlockSpec((N // 2, D), lambda i, b: (i, 0)),
        ],
        out_specs=pl.BlockSpec((1, N // 2, D), lambda i, b: (b, i, 0)),
        out_shape=jax.ShapeDtypeStruct((B, N, D), jnp.float32),
    )(inputs, pos_table)
